# baseline (speedup 1.0000x reference)
import functools

import jax
import jax.numpy as jnp
from jax.experimental import pallas as pl
from jax.experimental.pallas import tpu as pltpu

BN_EPS = 1e-5
NORM_EPS = 1e-12
D1 = 256
D2 = 128
VMEM_LIMIT = 64 * 1024 * 1024
NC = 2


def _branch_kernel(fa_ref, fb_ref,
                   w1a_ref, w1b_ref, g1a_ref, g1b_ref, be1a_ref, be1b_ref,
                   w2a_ref, w2b_ref, g2a_ref, g2b_ref, be2a_ref, be2b_ref,
                   out_ref, h1_s, stats_s, w1bf_s, *, nc, ch, b):
    i = pl.program_id(0)
    j = pl.program_id(1)
    on_a = i == 0

    @pl.when(j == 0)
    def _init():
        stats_s[...] = jnp.zeros_like(stats_s)
        w1bf_s[...] = jnp.where(on_a, w1a_ref[...], w1b_ref[...]).astype(
            jnp.bfloat16)

    def chunk(x_ref):
        xc = x_ref[...].astype(jnp.bfloat16)
        h1c = jnp.dot(xc, w1bf_s[...],
                      preferred_element_type=jnp.float32)
        h1_s[pl.ds(j * ch, ch), :] = h1c
        stats_s[0:1, :] += jnp.sum(h1c, axis=0, keepdims=True)
        stats_s[1:2, :] += jnp.sum(h1c * h1c, axis=0, keepdims=True)

    @pl.when(on_a)
    def _a():
        chunk(fa_ref)

    @pl.when(jnp.logical_not(on_a))
    def _b():
        chunk(fb_ref)

    @pl.when(j == nc - 1)
    def _tail():
        g1 = jnp.where(on_a, g1a_ref[...], g1b_ref[...])
        be1 = jnp.where(on_a, be1a_ref[...], be1b_ref[...])
        w2 = jnp.where(on_a, w2a_ref[...], w2b_ref[...]).astype(jnp.bfloat16)
        g2 = jnp.where(on_a, g2a_ref[...], g2b_ref[...])
        be2 = jnp.where(on_a, be2a_ref[...], be2b_ref[...])

        inv_n = 1.0 / float(b)
        mu = stats_s[0:1, :] * inv_n
        var = stats_s[1:2, :] * inv_n - mu * mu
        scale = g1 * jax.lax.rsqrt(var + BN_EPS)
        shift = be1 - mu * scale
        a1 = jnp.maximum(h1_s[...] * scale + shift, 0.0).astype(jnp.bfloat16)
        h2 = jnp.dot(a1, w2, preferred_element_type=jnp.float32)
        mu2 = jnp.mean(h2, axis=0, keepdims=True)
        d2 = h2 - mu2
        var2 = jnp.mean(d2 * d2, axis=0, keepdims=True)
        scale2 = g2 * jax.lax.rsqrt(var2 + BN_EPS)
        h2 = h2 * scale2 + (be2 - mu2 * scale2)
        inv = jax.lax.rsqrt(jnp.sum(h2 * h2, axis=1, keepdims=True) + NORM_EPS)
        out_ref[...] = (h2 * inv).astype(out_ref.dtype)


def _logits_kernel(ha_ref, hb_ref, out_ref):
    out_ref[...] = jax.lax.dot_general(
        ha_ref[...], hb_ref[...],
        dimension_numbers=(((1,), (1,)), ((), ())),
        preferred_element_type=jnp.float32,
    ).astype(out_ref.dtype)


def kernel(f_a, f_b,
           a_w1, a_b1, a_g1, a_be1, a_w2, a_b2, a_g2, a_be2,
           b_w1, b_b1, b_g1, b_be1, b_w2, b_b2, b_g2, b_be2):
    B, H = f_a.shape
    ch = B // NC

    def resident_spec(shape):
        return pl.BlockSpec(shape, lambda i, j: (0,) * len(shape))

    def x_spec(branch):
        return pl.BlockSpec(
            (ch, H),
            lambda i, j: (jnp.where(i == branch, j, 0), 0))

    h_n = pl.pallas_call(
        functools.partial(_branch_kernel, nc=NC, ch=ch, b=B),
        out_shape=jax.ShapeDtypeStruct((2, B, D2), jnp.bfloat16),
        grid=(2, NC),
        in_specs=[x_spec(0), x_spec(1),
                  resident_spec((H, D1)), resident_spec((H, D1)),
                  resident_spec((1, D1)), resident_spec((1, D1)),
                  resident_spec((1, D1)), resident_spec((1, D1)),
                  resident_spec((D1, D2)), resident_spec((D1, D2)),
                  resident_spec((1, D2)), resident_spec((1, D2)),
                  resident_spec((1, D2)), resident_spec((1, D2))],
        out_specs=pl.BlockSpec((None, B, D2), lambda i, j: (i, 0, 0)),
        scratch_shapes=[pltpu.VMEM((B, D1), jnp.float32),
                        pltpu.VMEM((2, D1), jnp.float32),
                        pltpu.VMEM((H, D1), jnp.bfloat16)],
        compiler_params=pltpu.CompilerParams(
            dimension_semantics=("parallel", "arbitrary"),
            vmem_limit_bytes=VMEM_LIMIT),
    )(f_a, f_b, a_w1, b_w1, a_g1, b_g1, a_be1, b_be1,
      a_w2, b_w2, a_g2, b_g2, a_be2, b_be2)

    tm = 1024 if B % 1024 == 0 else (256 if B % 256 == 0 else B)
    grid_m = pl.cdiv(B, tm)

    return pl.pallas_call(
        _logits_kernel,
        out_shape=jax.ShapeDtypeStruct((B, B), jnp.float32),
        grid=(grid_m,),
        in_specs=[pl.BlockSpec((None, tm, D2), lambda i: (0, i, 0)),
                  pl.BlockSpec((None, B, D2), lambda i: (1, 0, 0))],
        out_specs=pl.BlockSpec((tm, B), lambda i: (i, 0)),
        compiler_params=pltpu.CompilerParams(
            dimension_semantics=("parallel",),
            vmem_limit_bytes=VMEM_LIMIT),
    )(h_n, h_n)


# speedup vs baseline: 1.0069x; 1.0069x over previous
import functools

import jax
import jax.numpy as jnp
from jax.experimental import pallas as pl
from jax.experimental.pallas import tpu as pltpu

BN_EPS = 1e-5
NORM_EPS = 1e-12
D1 = 256
D2 = 128
VMEM_LIMIT = 64 * 1024 * 1024
NC = 2


def _branch_kernel(fa_ref, fb_ref,
                   w1a_ref, w1b_ref, g1a_ref, g1b_ref, be1a_ref, be1b_ref,
                   w2a_ref, w2b_ref, g2a_ref, g2b_ref, be2a_ref, be2b_ref,
                   out_ref, h1_s, stats_s, w1bf_s, *, nc, ch, b):
    i = pl.program_id(0)
    j = pl.program_id(1)
    on_a = i == 0

    @pl.when(j == 0)
    def _init():
        stats_s[...] = jnp.zeros_like(stats_s)
        w1bf_s[...] = jnp.where(on_a, w1a_ref[...], w1b_ref[...]).astype(
            jnp.bfloat16)

    def chunk(x_ref):
        xc = x_ref[...].astype(jnp.bfloat16)
        h1c = jnp.dot(xc, w1bf_s[...],
                      preferred_element_type=jnp.float32)
        h1_s[pl.ds(j * ch, ch), :] = h1c
        stats_s[0:1, :] += jnp.sum(h1c, axis=0, keepdims=True)
        stats_s[1:2, :] += jnp.sum(h1c * h1c, axis=0, keepdims=True)

    @pl.when(on_a)
    def _a():
        chunk(fa_ref)

    @pl.when(jnp.logical_not(on_a))
    def _b():
        chunk(fb_ref)

    @pl.when(j == nc - 1)
    def _tail():
        g1 = jnp.where(on_a, g1a_ref[...], g1b_ref[...])
        be1 = jnp.where(on_a, be1a_ref[...], be1b_ref[...])
        w2 = jnp.where(on_a, w2a_ref[...], w2b_ref[...]).astype(jnp.bfloat16)
        g2 = jnp.where(on_a, g2a_ref[...], g2b_ref[...])
        be2 = jnp.where(on_a, be2a_ref[...], be2b_ref[...])

        inv_n = 1.0 / float(b)
        mu = stats_s[0:1, :] * inv_n
        var = stats_s[1:2, :] * inv_n - mu * mu
        scale = g1 * jax.lax.rsqrt(var + BN_EPS)
        shift = be1 - mu * scale
        a1 = jnp.maximum(h1_s[...] * scale + shift, 0.0).astype(jnp.bfloat16)
        h2 = jnp.dot(a1, w2, preferred_element_type=jnp.float32)
        mu2 = jnp.mean(h2, axis=0, keepdims=True)
        d2 = h2 - mu2
        var2 = jnp.mean(d2 * d2, axis=0, keepdims=True)
        scale2 = g2 * jax.lax.rsqrt(var2 + BN_EPS)
        h2 = h2 * scale2 + (be2 - mu2 * scale2)
        inv = jax.lax.rsqrt(jnp.sum(h2 * h2, axis=1, keepdims=True) + NORM_EPS)
        out_ref[...] = (h2 * inv).astype(out_ref.dtype)


def _logits_kernel(ha_ref, hb_ref, out_ref):
    out_ref[...] = jax.lax.dot_general(
        ha_ref[...], hb_ref[...],
        dimension_numbers=(((1,), (1,)), ((), ())),
        preferred_element_type=jnp.float32,
    ).astype(out_ref.dtype)


def kernel(f_a, f_b,
           a_w1, a_b1, a_g1, a_be1, a_w2, a_b2, a_g2, a_be2,
           b_w1, b_b1, b_g1, b_be1, b_w2, b_b2, b_g2, b_be2):
    B, H = f_a.shape
    ch = B // NC

    def resident_spec(shape):
        return pl.BlockSpec(shape, lambda i, j: (0,) * len(shape))

    def x_spec(branch):
        return pl.BlockSpec(
            (ch, H),
            lambda i, j: (jnp.where(i == branch, j, 0), 0))

    h_n = pl.pallas_call(
        functools.partial(_branch_kernel, nc=NC, ch=ch, b=B),
        out_shape=jax.ShapeDtypeStruct((2, B, D2), jnp.bfloat16),
        grid=(2, NC),
        in_specs=[x_spec(0), x_spec(1),
                  resident_spec((H, D1)), resident_spec((H, D1)),
                  resident_spec((1, D1)), resident_spec((1, D1)),
                  resident_spec((1, D1)), resident_spec((1, D1)),
                  resident_spec((D1, D2)), resident_spec((D1, D2)),
                  resident_spec((1, D2)), resident_spec((1, D2)),
                  resident_spec((1, D2)), resident_spec((1, D2))],
        out_specs=pl.BlockSpec((None, B, D2), lambda i, j: (i, 0, 0)),
        scratch_shapes=[pltpu.VMEM((B, D1), jnp.float32),
                        pltpu.VMEM((2, D1), jnp.float32),
                        pltpu.VMEM((H, D1), jnp.bfloat16)],
        compiler_params=pltpu.CompilerParams(
            dimension_semantics=("parallel", "arbitrary"),
            vmem_limit_bytes=VMEM_LIMIT),
    )(f_a, f_b, a_w1, b_w1, a_g1, b_g1, a_be1, b_be1,
      a_w2, b_w2, a_g2, b_g2, a_be2, b_be2)

    tm = 256 if B % 256 == 0 else B
    grid_m = pl.cdiv(B, tm)

    return pl.pallas_call(
        _logits_kernel,
        out_shape=jax.ShapeDtypeStruct((B, B), jnp.float32),
        grid=(grid_m,),
        in_specs=[pl.BlockSpec((None, tm, D2), lambda i: (0, i, 0)),
                  pl.BlockSpec((None, B, D2), lambda i: (1, 0, 0))],
        out_specs=pl.BlockSpec((tm, B), lambda i: (i, 0)),
        compiler_params=pltpu.CompilerParams(
            dimension_semantics=("parallel",),
            vmem_limit_bytes=VMEM_LIMIT),
    )(h_n, h_n)


# speedup vs baseline: 1.0641x; 1.0568x over previous
import functools

import jax
import jax.numpy as jnp
from jax.experimental import pallas as pl
from jax.experimental.pallas import tpu as pltpu

BN_EPS = 1e-5
NORM_EPS = 1e-12
D1 = 256
D2 = 128
VMEM_LIMIT = 64 * 1024 * 1024
NC = 1


def _branch_kernel(fa_ref, fb_ref,
                   w1a_ref, w1b_ref, g1a_ref, g1b_ref, be1a_ref, be1b_ref,
                   w2a_ref, w2b_ref, g2a_ref, g2b_ref, be2a_ref, be2b_ref,
                   out_ref, h1_s, stats_s, w1bf_s, *, nc, ch, b):
    i = pl.program_id(0)
    j = pl.program_id(1)
    on_a = i == 0

    @pl.when(j == 0)
    def _init():
        stats_s[...] = jnp.zeros_like(stats_s)
        w1bf_s[...] = jnp.where(on_a, w1a_ref[...], w1b_ref[...]).astype(
            jnp.bfloat16)

    def chunk(x_ref):
        xc = x_ref[...].astype(jnp.bfloat16)
        h1c = jnp.dot(xc, w1bf_s[...],
                      preferred_element_type=jnp.float32)
        h1_s[pl.ds(j * ch, ch), :] = h1c
        stats_s[0:1, :] += jnp.sum(h1c, axis=0, keepdims=True)
        stats_s[1:2, :] += jnp.sum(h1c * h1c, axis=0, keepdims=True)

    @pl.when(on_a)
    def _a():
        chunk(fa_ref)

    @pl.when(jnp.logical_not(on_a))
    def _b():
        chunk(fb_ref)

    @pl.when(j == nc - 1)
    def _tail():
        g1 = jnp.where(on_a, g1a_ref[...], g1b_ref[...])
        be1 = jnp.where(on_a, be1a_ref[...], be1b_ref[...])
        w2 = jnp.where(on_a, w2a_ref[...], w2b_ref[...]).astype(jnp.bfloat16)
        g2 = jnp.where(on_a, g2a_ref[...], g2b_ref[...])
        be2 = jnp.where(on_a, be2a_ref[...], be2b_ref[...])

        inv_n = 1.0 / float(b)
        mu = stats_s[0:1, :] * inv_n
        var = stats_s[1:2, :] * inv_n - mu * mu
        scale = g1 * jax.lax.rsqrt(var + BN_EPS)
        shift = be1 - mu * scale
        a1 = jnp.maximum(h1_s[...] * scale + shift, 0.0).astype(jnp.bfloat16)
        h2 = jnp.dot(a1, w2, preferred_element_type=jnp.float32)
        mu2 = jnp.mean(h2, axis=0, keepdims=True)
        d2 = h2 - mu2
        var2 = jnp.mean(d2 * d2, axis=0, keepdims=True)
        scale2 = g2 * jax.lax.rsqrt(var2 + BN_EPS)
        h2 = h2 * scale2 + (be2 - mu2 * scale2)
        inv = jax.lax.rsqrt(jnp.sum(h2 * h2, axis=1, keepdims=True) + NORM_EPS)
        out_ref[...] = (h2 * inv).astype(out_ref.dtype)


def _logits_kernel(ha_ref, hb_ref, out_ref):
    out_ref[...] = jax.lax.dot_general(
        ha_ref[...], hb_ref[...],
        dimension_numbers=(((1,), (1,)), ((), ())),
        preferred_element_type=jnp.float32,
    ).astype(out_ref.dtype)


def kernel(f_a, f_b,
           a_w1, a_b1, a_g1, a_be1, a_w2, a_b2, a_g2, a_be2,
           b_w1, b_b1, b_g1, b_be1, b_w2, b_b2, b_g2, b_be2):
    B, H = f_a.shape
    ch = B // NC

    def resident_spec(shape):
        return pl.BlockSpec(shape, lambda i, j: (0,) * len(shape))

    def x_spec(branch):
        return pl.BlockSpec(
            (ch, H),
            lambda i, j: (jnp.where(i == branch, j, 0), 0))

    h_n = pl.pallas_call(
        functools.partial(_branch_kernel, nc=NC, ch=ch, b=B),
        out_shape=jax.ShapeDtypeStruct((2, B, D2), jnp.bfloat16),
        grid=(2, NC),
        in_specs=[x_spec(0), x_spec(1),
                  resident_spec((H, D1)), resident_spec((H, D1)),
                  resident_spec((1, D1)), resident_spec((1, D1)),
                  resident_spec((1, D1)), resident_spec((1, D1)),
                  resident_spec((D1, D2)), resident_spec((D1, D2)),
                  resident_spec((1, D2)), resident_spec((1, D2)),
                  resident_spec((1, D2)), resident_spec((1, D2))],
        out_specs=pl.BlockSpec((None, B, D2), lambda i, j: (i, 0, 0)),
        scratch_shapes=[pltpu.VMEM((B, D1), jnp.float32),
                        pltpu.VMEM((2, D1), jnp.float32),
                        pltpu.VMEM((H, D1), jnp.bfloat16)],
        compiler_params=pltpu.CompilerParams(
            dimension_semantics=("parallel", "arbitrary"),
            vmem_limit_bytes=VMEM_LIMIT),
    )(f_a, f_b, a_w1, b_w1, a_g1, b_g1, a_be1, b_be1,
      a_w2, b_w2, a_g2, b_g2, a_be2, b_be2)

    tm = 512 if B % 512 == 0 else (256 if B % 256 == 0 else B)
    grid_m = pl.cdiv(B, tm)

    return pl.pallas_call(
        _logits_kernel,
        out_shape=jax.ShapeDtypeStruct((B, B), jnp.float32),
        grid=(grid_m,),
        in_specs=[pl.BlockSpec((None, tm, D2), lambda i: (0, i, 0)),
                  pl.BlockSpec((None, B, D2), lambda i: (1, 0, 0))],
        out_specs=pl.BlockSpec((tm, B), lambda i: (i, 0)),
        compiler_params=pltpu.CompilerParams(
            dimension_semantics=("parallel",),
            vmem_limit_bytes=VMEM_LIMIT),
    )(h_n, h_n)
